# baseline (speedup 1.0000x reference)
import jax
import jax.numpy as jnp
from jax import lax
from jax.experimental import pallas as pl
from jax.experimental.pallas import tpu as pltpu
from jax.experimental.pallas import tpu_sc as plsc

NC = 2
NS = 16
NW = NC * NS

CHUNK = 80


def _proj_body(xr_ref, xi_ref, wt_ref, b_ref, out_ref):
    wt = wt_ref[...]
    b = b_ref[...]
    d = wt.shape[1]
    out_ref[:, 0:d] = jnp.dot(xr_ref[...], wt, preferred_element_type=jnp.float32) + b
    out_ref[:, d:2 * d] = jnp.dot(xi_ref[...], wt, preferred_element_type=jnp.float32) + b


def _projection(xr, xi, W, b, blk):
    n, d_in = xr.shape
    d_out = W.shape[0]
    grid = n // blk
    return pl.pallas_call(
        _proj_body,
        grid=(grid,),
        in_specs=[
            pl.BlockSpec((blk, d_in), lambda i: (i, 0)),
            pl.BlockSpec((blk, d_in), lambda i: (i, 0)),
            pl.BlockSpec((d_in, d_out), lambda i: (0, 0)),
            pl.BlockSpec((1, d_out), lambda i: (0, 0)),
        ],
        out_specs=pl.BlockSpec((blk, 2 * d_out), lambda i: (i, 0)),
        out_shape=jax.ShapeDtypeStruct((n, 2 * d_out), jnp.float32),
    )(xr, xi, W.T, b.reshape(1, d_out))


def _node_mlp_body(pn_ref, ns_ref, es_ref,
                   a_pn, a_ns, a_es, b0, alpha, wf, bf, out_ref, out_r, out_i):
    outs_split = (out_r, out_i)
    d = wf.shape[1]
    pn_full = pn_ref[...]
    ns_full = ns_ref[0] + ns_ref[1]
    es_full = es_ref[0] + es_ref[1]
    d_pn = pn_full.shape[1] // 2
    d_es = a_es.shape[0]
    for k in range(2):
        pn = pn_full[:, k * d_pn:(k + 1) * d_pn]
        nsum = ns_full[:, k * d_pn:(k + 1) * d_pn]
        esum = es_full[:, k * d_es:(k + 1) * d_es]
        h = (jnp.dot(pn, a_pn[...], preferred_element_type=jnp.float32)
             + jnp.dot(nsum, a_ns[...], preferred_element_type=jnp.float32)
             + jnp.dot(esum, a_es[...], preferred_element_type=jnp.float32)
             + b0[...])
        h = jnp.where(h >= 0, h, alpha[...] * h)
        o = jnp.dot(h, wf[...], preferred_element_type=jnp.float32) + bf[...]
        out_ref[:, k * d:(k + 1) * d] = o
        outs_split[k][...] = o


def _node_mlp(pn_c, ns_c, es_c, Wn0, bn0, an0, Wnf, bnf, blk):
    n = pn_c.shape[0]
    d_pn = pn_c.shape[1] // 2
    h_dim = Wn0.shape[0]
    d_es = h_dim - 2 * d_pn
    d_out = Wnf.shape[0]
    grid = n // blk
    W0t = Wn0.T
    a_pn = W0t[:d_pn]
    a_ns = W0t[d_pn:2 * d_pn]
    a_es = W0t[2 * d_pn:]
    return pl.pallas_call(
        _node_mlp_body,
        grid=(grid,),
        in_specs=[
            pl.BlockSpec((blk, 2 * d_pn), lambda i: (i, 0)),
            pl.BlockSpec((2, blk, 2 * d_pn), lambda i: (0, i, 0)),
            pl.BlockSpec((2, blk, es_c.shape[2]), lambda i: (0, i, 0)),
            pl.BlockSpec((d_pn, h_dim), lambda i: (0, 0)),
            pl.BlockSpec((d_pn, h_dim), lambda i: (0, 0)),
            pl.BlockSpec((d_es, h_dim), lambda i: (0, 0)),
            pl.BlockSpec((1, h_dim), lambda i: (0, 0)),
            pl.BlockSpec((1, 1), lambda i: (0, 0)),
            pl.BlockSpec((h_dim, d_out), lambda i: (0, 0)),
            pl.BlockSpec((1, d_out), lambda i: (0, 0)),
        ],
        out_specs=[
            pl.BlockSpec((blk, 2 * d_out), lambda i: (i, 0)),
            pl.BlockSpec((blk, d_out), lambda i: (i, 0)),
            pl.BlockSpec((blk, d_out), lambda i: (i, 0)),
        ],
        out_shape=[
            jax.ShapeDtypeStruct((n, 2 * d_out), jnp.float32),
            jax.ShapeDtypeStruct((n, d_out), jnp.float32),
            jax.ShapeDtypeStruct((n, d_out), jnp.float32),
        ],
    )(pn_c, ns_c, es_c,
      a_pn, a_ns, a_es, bn0.reshape(1, h_dim), an0.reshape(1, 1),
      Wnf.T, bnf.reshape(1, d_out))


def _edge_mlp_body(pe_ref, vi_ref, vj_ref,
                   b_pe, b_vi, b_vj, b0, alpha, wf, bf, out_r, out_i):
    bft = jnp.bfloat16
    pe_full = pe_ref[...].astype(bft)
    vi_full = vi_ref[...].astype(bft)
    vj_full = vj_ref[...].astype(bft)
    d_pe = pe_full.shape[1] // 2
    d_v = vi_full.shape[1] // 2
    for k, out in enumerate((out_r, out_i)):
        pe = pe_full[:, k * d_pe:(k + 1) * d_pe]
        vi = vi_full[:, k * d_v:(k + 1) * d_v]
        vj = vj_full[:, k * d_v:(k + 1) * d_v]
        g = (jnp.dot(pe, b_pe[...].astype(bft), preferred_element_type=jnp.float32)
             + jnp.dot(vi, b_vi[...].astype(bft), preferred_element_type=jnp.float32)
             + jnp.dot(vj, b_vj[...].astype(bft), preferred_element_type=jnp.float32)
             + b0[...])
        g = jnp.where(g >= 0, g, alpha[...] * g)
        out[...] = jnp.dot(g, wf[...], preferred_element_type=jnp.float32) + bf[...]


def _edge_mlp(pe_c, vi_c, vj_c, We0, be0, ae0, Wef, bef, blk):
    e = pe_c.shape[0]
    d_pe = pe_c.shape[1] // 2
    d_v = vi_c.shape[1] // 2
    h_dim = We0.shape[0]
    d_out = Wef.shape[0]
    grid = e // blk
    W0t = We0.T
    b_pe = W0t[:d_pe]
    b_vi = W0t[d_pe:d_pe + d_v]
    b_vj = W0t[d_pe + d_v:]
    out_sds = jax.ShapeDtypeStruct((e, d_out), jnp.float32)
    return pl.pallas_call(
        _edge_mlp_body,
        grid=(grid,),
        in_specs=[
            pl.BlockSpec((blk, 2 * d_pe), lambda i: (i, 0)),
            pl.BlockSpec((blk, 2 * d_v), lambda i: (i, 0)),
            pl.BlockSpec((blk, 2 * d_v), lambda i: (i, 0)),
            pl.BlockSpec((d_pe, h_dim), lambda i: (0, 0)),
            pl.BlockSpec((d_v, h_dim), lambda i: (0, 0)),
            pl.BlockSpec((d_v, h_dim), lambda i: (0, 0)),
            pl.BlockSpec((1, h_dim), lambda i: (0, 0)),
            pl.BlockSpec((1, 1), lambda i: (0, 0)),
            pl.BlockSpec((h_dim, d_out), lambda i: (0, 0)),
            pl.BlockSpec((1, d_out), lambda i: (0, 0)),
        ],
        out_specs=[
            pl.BlockSpec((blk, d_out), lambda i: (i, 0)),
            pl.BlockSpec((blk, d_out), lambda i: (i, 0)),
        ],
        out_shape=[out_sds, out_sds],
    )(pe_c, vi_c, vj_c,
      b_pe, b_vi, b_vj, be0.reshape(1, h_dim), ae0.reshape(1, 1),
      Wef.T, bef.reshape(1, d_out))


def _sc_mesh():
    return plsc.VectorSubcoreMesh(core_axis_name="c", subcore_axis_name="s",
                                  num_cores=NC, num_subcores=NS)


_DNUMS = lax.GatherDimensionNumbers(
    offset_dims=(), collapsed_slice_dims=(0,), start_index_map=(0,))


def _dein_parts():
    lane = lax.iota(jnp.int32, 16)
    gidx_e = ((2 * lane) % 16)[:, None]
    gidx_o = ((2 * lane + 1) % 16)[:, None]
    return lane, gidx_e, gidx_o


def _lane_pick(a, b, lane, gidx):
    ga = lax.gather(a, gidx, dimension_numbers=_DNUMS, slice_sizes=(1,),
                    mode=lax.GatherScatterMode.PROMISE_IN_BOUNDS)
    gb = lax.gather(b, gidx, dimension_numbers=_DNUMS, slice_sizes=(1,),
                    mode=lax.GatherScatterMode.PROMISE_IN_BOUNDS)
    return jnp.where(lane < 8, ga, gb)


def _node_seg_sum_sc(ei2, pn_c, n_pad, rows_per_sub, e):
    ew = e // NW
    nchunk = ew // CHUNK
    d_n = pn_c.shape[1]
    zrows = rows_per_sub // 8

    def body(ei_hbm, pn_hbm, ns_hbm,
             rc0, rc1, cc0, cc1, grow0, grow1, zbuf_n, pairs_v, acc_n,
             gsem0, gsem1):
        cid = lax.axis_index("c")
        sid = lax.axis_index("s")
        wid = sid * NC + cid
        rcs, ccs = (rc0, rc1), (cc0, cc1)
        grows = (grow0, grow1)
        gsems = (gsem0, gsem1)
        lane, gidx_e, gidx_o = _dein_parts()

        zero16 = jnp.zeros((16,), jnp.float32)
        for r in range(8):
            for cc in range(d_n // 16):
                zbuf_n[r, pl.ds(cc * 16, 16)] = zero16
        r0 = sid * rows_per_sub

        def zcopy(z, _):
            pltpu.sync_copy(zbuf_n, acc_n.at[pl.ds(r0 + z * 8, 8)])
            return 0

        lax.fori_loop(0, zrows, zcopy, 0)
        plsc.subcore_barrier()

        pltpu.sync_copy(ei_hbm.at[wid], pairs_v)

        def dein(j, p):
            base = pl.multiple_of(j * 2 * CHUNK, 32)
            for t in range(CHUNK // 16):
                a = pairs_v[pl.ds(base + t * 32, 16)]
                b = pairs_v[pl.ds(base + t * 32 + 16, 16)]
                rcs[p][pl.ds(t * 16, 16)] = _lane_pick(a, b, lane, gidx_e)
                ccs[p][pl.ds(t * 16, 16)] = _lane_pick(a, b, lane, gidx_o)

        def gather_fire(p):
            pltpu.async_copy(pn_hbm.at[ccs[p]], grows[p], gsems[p])

        for p in range(2):
            dein(p, p)
            gather_fire(p)

        def group(g, _):
            for p in range(2):
                j = g * 2 + p
                pltpu.make_async_copy(pn_hbm.at[pl.ds(0, CHUNK)],
                                      grows[p], gsems[p]).wait()
                pltpu.sync_copy(grows[p], acc_n.at[rcs[p]], add=True)

                @pl.when(j + 2 < nchunk)
                def _gf():
                    dein(j + 2, p)
                    gather_fire(p)

            return 0

        lax.fori_loop(0, nchunk // 2, group, 0)
        for j in range(nchunk - nchunk % 2, nchunk):
            p = j % 2
            pltpu.make_async_copy(pn_hbm.at[pl.ds(0, CHUNK)],
                                  grows[p], gsems[p]).wait()
            pltpu.sync_copy(grows[p], acc_n.at[rcs[p]], add=True)
        plsc.subcore_barrier()

        pltpu.sync_copy(acc_n.at[pl.ds(r0, rows_per_sub)],
                        ns_hbm.at[cid, pl.ds(r0, rows_per_sub)])

    f = pl.kernel(
        body,
        out_type=jax.ShapeDtypeStruct((NC, n_pad, d_n), jnp.float32),
        mesh=_sc_mesh(),
        scratch_types=[
            pltpu.VMEM((CHUNK,), jnp.int32),
            pltpu.VMEM((CHUNK,), jnp.int32),
            pltpu.VMEM((CHUNK,), jnp.int32),
            pltpu.VMEM((CHUNK,), jnp.int32),
            pltpu.VMEM((CHUNK, d_n), jnp.float32),
            pltpu.VMEM((CHUNK, d_n), jnp.float32),
            pltpu.VMEM((8, d_n), jnp.float32),
            pltpu.VMEM((2 * ew,), jnp.int32),
            pltpu.VMEM_SHARED((n_pad, d_n), jnp.float32),
            pltpu.SemaphoreType.DMA,
            pltpu.SemaphoreType.DMA,
        ],
    )
    return f(ei2, pn_c)


def _edge_seg_sum_sc(ei2, pe_c, n_pad, rows_per_sub):
    e = pe_c.shape[0]
    ew = e // NW
    nchunk = ew // CHUNK
    d_e = pe_c.shape[1]
    d_w = 128

    def body(ei_hbm, pe_hbm, es_hbm,
             rc0, rc1, pe_s, pe_v, zbuf_e, pairs_v, acc_e, esem):
        cid = lax.axis_index("c")
        sid = lax.axis_index("s")
        wid = sid * NC + cid
        rcs = (rc0, rc1)
        lane, gidx_e, _go = _dein_parts()

        zero16 = jnp.zeros((16,), jnp.float32)
        for r in range(8):
            for cc in range(d_w // 16):
                zbuf_e[r, pl.ds(cc * 16, 16)] = zero16
        for r in range(CHUNK):
            for cc in range(d_w // 16):
                pe_v[r, pl.ds(cc * 16, 16)] = zero16
        r0 = sid * rows_per_sub

        def zcopy(z, _):
            pltpu.sync_copy(zbuf_e, acc_e.at[pl.ds(r0 + z * 8, 8)])
            return 0

        lax.fori_loop(0, rows_per_sub // 8, zcopy, 0)
        plsc.subcore_barrier()

        ebase = wid * ew
        pltpu.sync_copy(ei_hbm.at[wid], pairs_v)

        def dein(j, p):
            base = pl.multiple_of(j * 2 * CHUNK, 32)
            for t in range(CHUNK // 16):
                a = pairs_v[pl.ds(base + t * 32, 16)]
                b = pairs_v[pl.ds(base + t * 32 + 16, 16)]
                rcs[p][pl.ds(t * 16, 16)] = _lane_pick(a, b, lane, gidx_e)

        def pe_fire(j):
            pltpu.async_copy(pe_hbm.at[pl.ds(ebase + j * CHUNK, CHUNK)],
                             pe_s, esem)

        dein(0, 0)
        pe_fire(0)

        def step(j, p):
            pltpu.make_async_copy(pe_hbm.at[pl.ds(0, CHUNK)],
                                  pe_s, esem).wait()
            for r in range(CHUNK):
                for t in range(d_e // 16):
                    pe_v[r, pl.ds(t * 16, 16)] = pe_s[r, pl.ds(t * 16, 16)]

            @pl.when(j + 1 < nchunk)
            def _nf():
                pe_fire(j + 1)
                dein(j + 1, 1 - p)

            pltpu.sync_copy(pe_v, acc_e.at[rcs[p]], add=True)
            return 0

        def group(g, _):
            step(2 * g, 0)
            step(2 * g + 1, 1)
            return 0

        lax.fori_loop(0, nchunk // 2, group, 0)
        for j in range(nchunk - nchunk % 2, nchunk):
            step(j, j % 2)
        plsc.subcore_barrier()

        pltpu.sync_copy(acc_e.at[pl.ds(r0, rows_per_sub)],
                        es_hbm.at[cid, pl.ds(r0, rows_per_sub)])

    f = pl.kernel(
        body,
        out_type=jax.ShapeDtypeStruct((NC, n_pad, d_w), jnp.float32),
        mesh=_sc_mesh(),
        scratch_types=[
            pltpu.VMEM((CHUNK,), jnp.int32),
            pltpu.VMEM((CHUNK,), jnp.int32),
            pltpu.VMEM((CHUNK, d_e), jnp.float32),
            pltpu.VMEM((CHUNK, d_w), jnp.float32),
            pltpu.VMEM((8, d_w), jnp.float32),
            pltpu.VMEM((2 * ew,), jnp.int32),
            pltpu.VMEM_SHARED((n_pad, d_w), jnp.float32),
            pltpu.SemaphoreType.DMA,
        ],
    )
    return f(ei2, pe_c)


def _edge_gather_sc(ei2, no_c, e):
    ew = e // NW
    nchunk = ew // CHUNK
    d_n = no_c.shape[1]
    dt = no_c.dtype

    def body(ei_hbm, no_hbm, vi_hbm, vj_hbm,
             rc0, rc1, cc0, cc1, a0, a1, b0, b1, pairs_v,
             sa0, sa1, sb0, sb1):
        cid = lax.axis_index("c")
        sid = lax.axis_index("s")
        wid = sid * NC + cid
        rcs, ccs = (rc0, rc1), (cc0, cc1)
        abufs, bbufs = (a0, a1), (b0, b1)
        asems, bsems = (sa0, sa1), (sb0, sb1)
        lane, gidx_e, gidx_o = _dein_parts()
        ebase = wid * ew

        pltpu.sync_copy(ei_hbm.at[wid], pairs_v)

        def dein(j, p):
            base = pl.multiple_of(j * 2 * CHUNK, 32)
            for t in range(CHUNK // 16):
                a = pairs_v[pl.ds(base + t * 32, 16)]
                b = pairs_v[pl.ds(base + t * 32 + 16, 16)]
                rcs[p][pl.ds(t * 16, 16)] = _lane_pick(a, b, lane, gidx_e)
                ccs[p][pl.ds(t * 16, 16)] = _lane_pick(a, b, lane, gidx_o)

        def gather_fire(p):
            pltpu.async_copy(no_hbm.at[rcs[p]], abufs[p], asems[p])
            pltpu.async_copy(no_hbm.at[ccs[p]], bbufs[p], bsems[p])

        for p in range(2):
            dein(p, p)
            gather_fire(p)

        def drain_write(j, p):
            dst = pl.ds(ebase + j * CHUNK, CHUNK)
            pltpu.make_async_copy(no_hbm.at[pl.ds(0, CHUNK)],
                                  abufs[p], asems[p]).wait()
            pltpu.sync_copy(abufs[p], vi_hbm.at[dst])
            pltpu.make_async_copy(no_hbm.at[pl.ds(0, CHUNK)],
                                  bbufs[p], bsems[p]).wait()
            pltpu.sync_copy(bbufs[p], vj_hbm.at[dst])

        def group(g, _):
            for p in range(2):
                j = g * 2 + p
                drain_write(j, p)

                @pl.when(j + 2 < nchunk)
                def _gf():
                    dein(j + 2, p)
                    gather_fire(p)

            return 0

        lax.fori_loop(0, nchunk // 2, group, 0)
        for j in range(nchunk - nchunk % 2, nchunk):
            drain_write(j, j % 2)

    out_sds = jax.ShapeDtypeStruct((e, d_n), dt)
    f = pl.kernel(
        body,
        out_type=(out_sds, out_sds),
        mesh=_sc_mesh(),
        scratch_types=[
            pltpu.VMEM((CHUNK,), jnp.int32),
            pltpu.VMEM((CHUNK,), jnp.int32),
            pltpu.VMEM((CHUNK,), jnp.int32),
            pltpu.VMEM((CHUNK,), jnp.int32),
            pltpu.VMEM((CHUNK, d_n), dt),
            pltpu.VMEM((CHUNK, d_n), dt),
            pltpu.VMEM((CHUNK, d_n), dt),
            pltpu.VMEM((CHUNK, d_n), dt),
            pltpu.VMEM((2 * ew,), jnp.int32),
            pltpu.SemaphoreType.DMA,
            pltpu.SemaphoreType.DMA,
            pltpu.SemaphoreType.DMA,
            pltpu.SemaphoreType.DMA,
        ],
    )
    return f(ei2, no_c)


def kernel(node_feats_real, node_feats_imag, edge_feats_real, edge_feats_imag,
           edge_index, Wpn, bpn, Wpe, bpe, Wn0, bn0, an0, Wnf, bnf,
           We0, be0, ae0, Wef, bef):
    n = node_feats_real.shape[0]
    e = edge_feats_real.shape[0]
    d_out_node = Wnf.shape[0]

    ew = e // NW
    assert ew * NW == e and ew % CHUNK == 0
    nchunk = ew // CHUNK
    rows_per_sub = (-(-n // NS) + 7) // 8 * 8
    n_pad = rows_per_sub * NS

    ei2 = edge_index.reshape(NW, 2 * ew)

    pn_c = _projection(node_feats_real, node_feats_imag, Wpn, bpn, blk=1000)
    pe_c = _projection(edge_feats_real, edge_feats_imag, Wpe, bpe, blk=4000)

    ns_c = _node_seg_sum_sc(ei2, pn_c, n_pad, rows_per_sub, e)
    es_c = _edge_seg_sum_sc(ei2, pe_c, n_pad, rows_per_sub)

    no_c, no_r, no_i = _node_mlp(pn_c, ns_c, es_c, Wn0, bn0, an0, Wnf, bnf,
                                 blk=1000)

    vi_c, vj_c = _edge_gather_sc(ei2, no_c, e)

    eo_r, eo_i = _edge_mlp(pe_c, vi_c, vj_c, We0, be0, ae0, Wef, bef, blk=4000)

    return no_r, no_i, eo_r, eo_i


# speedup vs baseline: 1.1727x; 1.1727x over previous
import jax
import jax.numpy as jnp
from jax import lax
from jax.experimental import pallas as pl
from jax.experimental.pallas import tpu as pltpu
from jax.experimental.pallas import tpu_sc as plsc

NC = 2
NS = 16
NW = NC * NS

CHUNK = 80


def _proj_body(xr_ref, xi_ref, wt_ref, b_ref, out_ref):
    wt = wt_ref[...]
    b = b_ref[...]
    d = wt.shape[1]
    out_ref[:, 0:d] = jnp.dot(xr_ref[...], wt, preferred_element_type=jnp.float32) + b
    out_ref[:, d:2 * d] = jnp.dot(xi_ref[...], wt, preferred_element_type=jnp.float32) + b


def _projection(xr, xi, W, b, blk):
    n, d_in = xr.shape
    d_out = W.shape[0]
    grid = n // blk
    return pl.pallas_call(
        _proj_body,
        grid=(grid,),
        in_specs=[
            pl.BlockSpec((blk, d_in), lambda i: (i, 0)),
            pl.BlockSpec((blk, d_in), lambda i: (i, 0)),
            pl.BlockSpec((d_in, d_out), lambda i: (0, 0)),
            pl.BlockSpec((1, d_out), lambda i: (0, 0)),
        ],
        out_specs=pl.BlockSpec((blk, 2 * d_out), lambda i: (i, 0)),
        out_shape=jax.ShapeDtypeStruct((n, 2 * d_out), jnp.float32),
    )(xr, xi, W.T, b.reshape(1, d_out))


def _node_mlp_body(pn_ref, ns_ref, es_ref,
                   a_pn, a_ns, a_es, b0, alpha, wf, bf, out_ref, out_r, out_i):
    outs_split = (out_r, out_i)
    d = wf.shape[1]
    pn_full = pn_ref[...]
    ns_full = ns_ref[0] + ns_ref[1]
    es_full = es_ref[0] + es_ref[1]
    d_pn = pn_full.shape[1] // 2
    d_es = a_es.shape[0]
    for k in range(2):
        pn = pn_full[:, k * d_pn:(k + 1) * d_pn]
        nsum = ns_full[:, k * d_pn:(k + 1) * d_pn]
        esum = es_full[:, k * d_es:(k + 1) * d_es]
        h = (jnp.dot(pn, a_pn[...], preferred_element_type=jnp.float32)
             + jnp.dot(nsum, a_ns[...], preferred_element_type=jnp.float32)
             + jnp.dot(esum, a_es[...], preferred_element_type=jnp.float32)
             + b0[...])
        h = jnp.where(h >= 0, h, alpha[...] * h)
        o = jnp.dot(h, wf[...], preferred_element_type=jnp.float32) + bf[...]
        out_ref[:, k * d:(k + 1) * d] = o
        outs_split[k][...] = o


def _node_mlp(pn_c, ns_c, es_c, Wn0, bn0, an0, Wnf, bnf, blk):
    n = pn_c.shape[0]
    d_pn = pn_c.shape[1] // 2
    h_dim = Wn0.shape[0]
    d_es = h_dim - 2 * d_pn
    d_out = Wnf.shape[0]
    grid = n // blk
    W0t = Wn0.T
    a_pn = W0t[:d_pn]
    a_ns = W0t[d_pn:2 * d_pn]
    a_es = W0t[2 * d_pn:]
    return pl.pallas_call(
        _node_mlp_body,
        grid=(grid,),
        in_specs=[
            pl.BlockSpec((blk, 2 * d_pn), lambda i: (i, 0)),
            pl.BlockSpec((2, blk, 2 * d_pn), lambda i: (0, i, 0)),
            pl.BlockSpec((2, blk, es_c.shape[2]), lambda i: (0, i, 0)),
            pl.BlockSpec((d_pn, h_dim), lambda i: (0, 0)),
            pl.BlockSpec((d_pn, h_dim), lambda i: (0, 0)),
            pl.BlockSpec((d_es, h_dim), lambda i: (0, 0)),
            pl.BlockSpec((1, h_dim), lambda i: (0, 0)),
            pl.BlockSpec((1, 1), lambda i: (0, 0)),
            pl.BlockSpec((h_dim, d_out), lambda i: (0, 0)),
            pl.BlockSpec((1, d_out), lambda i: (0, 0)),
        ],
        out_specs=[
            pl.BlockSpec((blk, 2 * d_out), lambda i: (i, 0)),
            pl.BlockSpec((blk, d_out), lambda i: (i, 0)),
            pl.BlockSpec((blk, d_out), lambda i: (i, 0)),
        ],
        out_shape=[
            jax.ShapeDtypeStruct((n, 2 * d_out), jnp.float32),
            jax.ShapeDtypeStruct((n, d_out), jnp.float32),
            jax.ShapeDtypeStruct((n, d_out), jnp.float32),
        ],
    )(pn_c, ns_c, es_c,
      a_pn, a_ns, a_es, bn0.reshape(1, h_dim), an0.reshape(1, 1),
      Wnf.T, bnf.reshape(1, d_out))


def _edge_mlp_body(pe_ref, vi_ref, vj_ref,
                   b_pe, b_vi, b_vj, b0, alpha, wf, bf, out_r, out_i):
    bft = jnp.bfloat16
    pe_full = pe_ref[...].astype(bft)
    vi_full = vi_ref[...].astype(bft)
    vj_full = vj_ref[...].astype(bft)
    d_pe = pe_full.shape[1] // 2
    d_v = vi_full.shape[1] // 2
    for k, out in enumerate((out_r, out_i)):
        pe = pe_full[:, k * d_pe:(k + 1) * d_pe]
        vi = vi_full[:, k * d_v:(k + 1) * d_v]
        vj = vj_full[:, k * d_v:(k + 1) * d_v]
        g = (jnp.dot(pe, b_pe[...].astype(bft), preferred_element_type=jnp.float32)
             + jnp.dot(vi, b_vi[...].astype(bft), preferred_element_type=jnp.float32)
             + jnp.dot(vj, b_vj[...].astype(bft), preferred_element_type=jnp.float32)
             + b0[...])
        g = jnp.where(g >= 0, g, alpha[...] * g)
        out[...] = jnp.dot(g, wf[...], preferred_element_type=jnp.float32) + bf[...]


def _edge_mlp(pe_c, vi_c, vj_c, We0, be0, ae0, Wef, bef, blk):
    e = pe_c.shape[0]
    d_pe = pe_c.shape[1] // 2
    d_v = vi_c.shape[1] // 2
    h_dim = We0.shape[0]
    d_out = Wef.shape[0]
    grid = e // blk
    W0t = We0.T
    b_pe = W0t[:d_pe]
    b_vi = W0t[d_pe:d_pe + d_v]
    b_vj = W0t[d_pe + d_v:]
    out_sds = jax.ShapeDtypeStruct((e, d_out), jnp.float32)
    return pl.pallas_call(
        _edge_mlp_body,
        grid=(grid,),
        in_specs=[
            pl.BlockSpec((blk, 2 * d_pe), lambda i: (i, 0)),
            pl.BlockSpec((blk, 2 * d_v), lambda i: (i, 0)),
            pl.BlockSpec((blk, 2 * d_v), lambda i: (i, 0)),
            pl.BlockSpec((d_pe, h_dim), lambda i: (0, 0)),
            pl.BlockSpec((d_v, h_dim), lambda i: (0, 0)),
            pl.BlockSpec((d_v, h_dim), lambda i: (0, 0)),
            pl.BlockSpec((1, h_dim), lambda i: (0, 0)),
            pl.BlockSpec((1, 1), lambda i: (0, 0)),
            pl.BlockSpec((h_dim, d_out), lambda i: (0, 0)),
            pl.BlockSpec((1, d_out), lambda i: (0, 0)),
        ],
        out_specs=[
            pl.BlockSpec((blk, d_out), lambda i: (i, 0)),
            pl.BlockSpec((blk, d_out), lambda i: (i, 0)),
        ],
        out_shape=[out_sds, out_sds],
    )(pe_c, vi_c, vj_c,
      b_pe, b_vi, b_vj, be0.reshape(1, h_dim), ae0.reshape(1, 1),
      Wef.T, bef.reshape(1, d_out))


def _sc_mesh():
    return plsc.VectorSubcoreMesh(core_axis_name="c", subcore_axis_name="s",
                                  num_cores=NC, num_subcores=NS)


_DNUMS = lax.GatherDimensionNumbers(
    offset_dims=(), collapsed_slice_dims=(0,), start_index_map=(0,))


def _dein_parts():
    lane = lax.iota(jnp.int32, 16)
    gidx_e = ((2 * lane) % 16)[:, None]
    gidx_o = ((2 * lane + 1) % 16)[:, None]
    return lane, gidx_e, gidx_o


def _lane_pick(a, b, lane, gidx):
    ga = lax.gather(a, gidx, dimension_numbers=_DNUMS, slice_sizes=(1,),
                    mode=lax.GatherScatterMode.PROMISE_IN_BOUNDS)
    gb = lax.gather(b, gidx, dimension_numbers=_DNUMS, slice_sizes=(1,),
                    mode=lax.GatherScatterMode.PROMISE_IN_BOUNDS)
    return jnp.where(lane < 8, ga, gb)


def _node_seg_sum_sc(row3, col2, pn_c, n_pad, rows_per_sub, e):
    ew = e // NW
    nchunk = ew // CHUNK
    d_n = pn_c.shape[1]
    zrows = rows_per_sub // 8

    def body(row_hbm, col_hbm, pn_hbm, ns_hbm,
             row_v, col_v, grow0, grow1, zbuf_n, acc_n,
             gsem0, gsem1):
        cid = lax.axis_index("c")
        sid = lax.axis_index("s")
        wid = sid * NC + cid
        grows = (grow0, grow1)
        gsems = (gsem0, gsem1)

        zero16 = jnp.zeros((16,), jnp.float32)
        for r in range(8):
            for cc in range(d_n // 16):
                zbuf_n[r, pl.ds(cc * 16, 16)] = zero16
        r0 = sid * rows_per_sub

        def zcopy(z, _):
            pltpu.sync_copy(zbuf_n, acc_n.at[pl.ds(r0 + z * 8, 8)])
            return 0

        lax.fori_loop(0, zrows, zcopy, 0)
        plsc.subcore_barrier()

        pltpu.sync_copy(row_hbm.at[wid], row_v)
        pltpu.sync_copy(col_hbm.at[wid], col_v)

        def gather_fire(j, p):
            idx = col_v.at[pl.ds(j * CHUNK, CHUNK)]
            pltpu.async_copy(pn_hbm.at[idx], grows[p], gsems[p])

        for p in range(2):
            gather_fire(p, p)

        def group(g, _):
            for p in range(2):
                j = g * 2 + p
                pltpu.make_async_copy(pn_hbm.at[pl.ds(0, CHUNK)],
                                      grows[p], gsems[p]).wait()
                pltpu.sync_copy(grows[p], acc_n.at[row_v.at[j]], add=True)

                @pl.when(j + 2 < nchunk)
                def _gf():
                    gather_fire(j + 2, p)

            return 0

        lax.fori_loop(0, nchunk // 2, group, 0)
        for j in range(nchunk - nchunk % 2, nchunk):
            p = j % 2
            pltpu.make_async_copy(pn_hbm.at[pl.ds(0, CHUNK)],
                                  grows[p], gsems[p]).wait()
            pltpu.sync_copy(grows[p], acc_n.at[row_v.at[j]], add=True)
        plsc.subcore_barrier()

        pltpu.sync_copy(acc_n.at[pl.ds(r0, rows_per_sub)],
                        ns_hbm.at[cid, pl.ds(r0, rows_per_sub)])

    f = pl.kernel(
        body,
        out_type=jax.ShapeDtypeStruct((NC, n_pad, d_n), jnp.float32),
        mesh=_sc_mesh(),
        scratch_types=[
            pltpu.VMEM((nchunk, CHUNK), jnp.int32),
            pltpu.VMEM((ew,), jnp.int32),
            pltpu.VMEM((CHUNK, d_n), jnp.float32),
            pltpu.VMEM((CHUNK, d_n), jnp.float32),
            pltpu.VMEM((8, d_n), jnp.float32),
            pltpu.VMEM_SHARED((n_pad, d_n), jnp.float32),
            pltpu.SemaphoreType.DMA,
            pltpu.SemaphoreType.DMA,
        ],
    )
    return f(row3, col2, pn_c)


def _edge_seg_sum_sc(row3, pe_c, n_pad, rows_per_sub):
    e = pe_c.shape[0]
    ew = e // NW
    nchunk = ew // CHUNK
    d_e = pe_c.shape[1]
    d_w = 128

    def body(row_hbm, pe_hbm, es_hbm,
             row_v, pe_s, pe_v, zbuf_e, acc_e, esem):
        cid = lax.axis_index("c")
        sid = lax.axis_index("s")
        wid = sid * NC + cid

        zero16 = jnp.zeros((16,), jnp.float32)
        for r in range(8):
            for cc in range(d_w // 16):
                zbuf_e[r, pl.ds(cc * 16, 16)] = zero16
        for r in range(CHUNK):
            for cc in range(d_w // 16):
                pe_v[r, pl.ds(cc * 16, 16)] = zero16
        r0 = sid * rows_per_sub

        def zcopy(z, _):
            pltpu.sync_copy(zbuf_e, acc_e.at[pl.ds(r0 + z * 8, 8)])
            return 0

        lax.fori_loop(0, rows_per_sub // 8, zcopy, 0)
        plsc.subcore_barrier()

        ebase = wid * ew
        pltpu.sync_copy(row_hbm.at[wid], row_v)

        def pe_fire(j):
            pltpu.async_copy(pe_hbm.at[pl.ds(ebase + j * CHUNK, CHUNK)],
                             pe_s, esem)

        pe_fire(0)

        def step(j, _):
            pltpu.make_async_copy(pe_hbm.at[pl.ds(0, CHUNK)],
                                  pe_s, esem).wait()
            for r in range(CHUNK):
                for t in range(d_e // 16):
                    pe_v[r, pl.ds(t * 16, 16)] = pe_s[r, pl.ds(t * 16, 16)]

            @pl.when(j + 1 < nchunk)
            def _nf():
                pe_fire(j + 1)

            pltpu.sync_copy(pe_v, acc_e.at[row_v.at[j]], add=True)
            return 0

        lax.fori_loop(0, nchunk, step, 0)
        plsc.subcore_barrier()

        pltpu.sync_copy(acc_e.at[pl.ds(r0, rows_per_sub)],
                        es_hbm.at[cid, pl.ds(r0, rows_per_sub)])

    f = pl.kernel(
        body,
        out_type=jax.ShapeDtypeStruct((NC, n_pad, d_w), jnp.float32),
        mesh=_sc_mesh(),
        scratch_types=[
            pltpu.VMEM((nchunk, CHUNK), jnp.int32),
            pltpu.VMEM((CHUNK, d_e), jnp.float32),
            pltpu.VMEM((CHUNK, d_w), jnp.float32),
            pltpu.VMEM((8, d_w), jnp.float32),
            pltpu.VMEM_SHARED((n_pad, d_w), jnp.float32),
            pltpu.SemaphoreType.DMA,
        ],
    )
    return f(row3, pe_c)


def _edge_gather_sc(row2, col2, no_c, e):
    ew = e // NW
    nchunk = ew // CHUNK
    d_n = no_c.shape[1]
    dt = no_c.dtype

    def body(row_hbm, col_hbm, no_hbm, vi_hbm, vj_hbm,
             row_v, col_v, a0, a1, b0, b1,
             sa0, sa1, sb0, sb1):
        cid = lax.axis_index("c")
        sid = lax.axis_index("s")
        wid = sid * NC + cid
        abufs, bbufs = (a0, a1), (b0, b1)
        asems, bsems = (sa0, sa1), (sb0, sb1)
        ebase = wid * ew

        pltpu.sync_copy(row_hbm.at[wid], row_v)
        pltpu.sync_copy(col_hbm.at[wid], col_v)

        def gather_fire(j, p):
            sl = pl.ds(j * CHUNK, CHUNK)
            pltpu.async_copy(no_hbm.at[row_v.at[sl]], abufs[p], asems[p])
            pltpu.async_copy(no_hbm.at[col_v.at[sl]], bbufs[p], bsems[p])

        for p in range(2):
            gather_fire(p, p)

        def drain_write(j, p):
            dst = pl.ds(ebase + j * CHUNK, CHUNK)
            pltpu.make_async_copy(no_hbm.at[pl.ds(0, CHUNK)],
                                  abufs[p], asems[p]).wait()
            pltpu.sync_copy(abufs[p], vi_hbm.at[dst])
            pltpu.make_async_copy(no_hbm.at[pl.ds(0, CHUNK)],
                                  bbufs[p], bsems[p]).wait()
            pltpu.sync_copy(bbufs[p], vj_hbm.at[dst])

        def group(g, _):
            for p in range(2):
                j = g * 2 + p
                drain_write(j, p)

                @pl.when(j + 2 < nchunk)
                def _gf():
                    gather_fire(j + 2, p)

            return 0

        lax.fori_loop(0, nchunk // 2, group, 0)
        for j in range(nchunk - nchunk % 2, nchunk):
            drain_write(j, j % 2)

    out_sds = jax.ShapeDtypeStruct((e, d_n), dt)
    f = pl.kernel(
        body,
        out_type=(out_sds, out_sds),
        mesh=_sc_mesh(),
        scratch_types=[
            pltpu.VMEM((ew,), jnp.int32),
            pltpu.VMEM((ew,), jnp.int32),
            pltpu.VMEM((CHUNK, d_n), dt),
            pltpu.VMEM((CHUNK, d_n), dt),
            pltpu.VMEM((CHUNK, d_n), dt),
            pltpu.VMEM((CHUNK, d_n), dt),
            pltpu.SemaphoreType.DMA,
            pltpu.SemaphoreType.DMA,
            pltpu.SemaphoreType.DMA,
            pltpu.SemaphoreType.DMA,
        ],
    )
    return f(row2, col2, no_c)


def kernel(node_feats_real, node_feats_imag, edge_feats_real, edge_feats_imag,
           edge_index, Wpn, bpn, Wpe, bpe, Wn0, bn0, an0, Wnf, bnf,
           We0, be0, ae0, Wef, bef):
    n = node_feats_real.shape[0]
    e = edge_feats_real.shape[0]
    d_out_node = Wnf.shape[0]

    ew = e // NW
    assert ew * NW == e and ew % CHUNK == 0
    nchunk = ew // CHUNK
    rows_per_sub = (-(-n // NS) + 7) // 8 * 8
    n_pad = rows_per_sub * NS

    row = edge_index[:, 0]
    col = edge_index[:, 1]
    row3 = row.reshape(NW, nchunk, CHUNK)
    row2 = row.reshape(NW, ew)
    col2 = col.reshape(NW, ew)

    pn_c = _projection(node_feats_real, node_feats_imag, Wpn, bpn, blk=1000)
    pe_c = _projection(edge_feats_real, edge_feats_imag, Wpe, bpe, blk=4000)

    ns_c = _node_seg_sum_sc(row3, col2, pn_c, n_pad, rows_per_sub, e)
    es_c = _edge_seg_sum_sc(row3, pe_c, n_pad, rows_per_sub)

    no_c, no_r, no_i = _node_mlp(pn_c, ns_c, es_c, Wn0, bn0, an0, Wnf, bnf,
                                 blk=1000)

    vi_c, vj_c = _edge_gather_sc(row2, col2, no_c, e)

    eo_r, eo_i = _edge_mlp(pe_c, vi_c, vj_c, We0, be0, ae0, Wef, bef, blk=4000)

    return no_r, no_i, eo_r, eo_i


# speedup vs baseline: 1.1844x; 1.0100x over previous
import jax
import jax.numpy as jnp
from jax import lax
from jax.experimental import pallas as pl
from jax.experimental.pallas import tpu as pltpu
from jax.experimental.pallas import tpu_sc as plsc

NC = 2
NS = 16
NW = NC * NS

CHUNK = 80


def _proj_body(xr_ref, xi_ref, wt_ref, b_ref, out_ref):
    wt = wt_ref[...]
    b = b_ref[...]
    d = wt.shape[1]
    out_ref[:, 0:d] = jnp.dot(xr_ref[...], wt, preferred_element_type=jnp.float32) + b
    out_ref[:, d:2 * d] = jnp.dot(xi_ref[...], wt, preferred_element_type=jnp.float32) + b


def _projection(xr, xi, W, b, blk):
    n, d_in = xr.shape
    d_out = W.shape[0]
    grid = n // blk
    return pl.pallas_call(
        _proj_body,
        grid=(grid,),
        in_specs=[
            pl.BlockSpec((blk, d_in), lambda i: (i, 0)),
            pl.BlockSpec((blk, d_in), lambda i: (i, 0)),
            pl.BlockSpec((d_in, d_out), lambda i: (0, 0)),
            pl.BlockSpec((1, d_out), lambda i: (0, 0)),
        ],
        out_specs=pl.BlockSpec((blk, 2 * d_out), lambda i: (i, 0)),
        out_shape=jax.ShapeDtypeStruct((n, 2 * d_out), jnp.float32),
    )(xr, xi, W.T, b.reshape(1, d_out))


def _node_mlp_body(pn_ref, ns_ref, es_ref,
                   a_pn, a_ns, a_es, b0, alpha, wf, bf, out_ref, out_r, out_i):
    outs_split = (out_r, out_i)
    d = wf.shape[1]
    pn_full = pn_ref[...]
    ns_full = ns_ref[0] + ns_ref[1]
    es_full = es_ref[0] + es_ref[1]
    d_pn = pn_full.shape[1] // 2
    d_es = a_es.shape[0]
    for k in range(2):
        pn = pn_full[:, k * d_pn:(k + 1) * d_pn]
        nsum = ns_full[:, k * d_pn:(k + 1) * d_pn]
        esum = es_full[:, k * d_es:(k + 1) * d_es]
        h = (jnp.dot(pn, a_pn[...], preferred_element_type=jnp.float32)
             + jnp.dot(nsum, a_ns[...], preferred_element_type=jnp.float32)
             + jnp.dot(esum, a_es[...], preferred_element_type=jnp.float32)
             + b0[...])
        h = jnp.where(h >= 0, h, alpha[...] * h)
        o = jnp.dot(h, wf[...], preferred_element_type=jnp.float32) + bf[...]
        out_ref[:, k * d:(k + 1) * d] = o
        outs_split[k][...] = o


def _node_mlp(pn_c, ns_c, es_c, Wn0, bn0, an0, Wnf, bnf, blk):
    n = pn_c.shape[0]
    d_pn = pn_c.shape[1] // 2
    h_dim = Wn0.shape[0]
    d_es = h_dim - 2 * d_pn
    d_out = Wnf.shape[0]
    grid = n // blk
    W0t = Wn0.T
    a_pn = W0t[:d_pn]
    a_ns = W0t[d_pn:2 * d_pn]
    a_es = W0t[2 * d_pn:]
    return pl.pallas_call(
        _node_mlp_body,
        grid=(grid,),
        in_specs=[
            pl.BlockSpec((blk, 2 * d_pn), lambda i: (i, 0)),
            pl.BlockSpec((2, blk, 2 * d_pn), lambda i: (0, i, 0)),
            pl.BlockSpec((2, blk, es_c.shape[2]), lambda i: (0, i, 0)),
            pl.BlockSpec((d_pn, h_dim), lambda i: (0, 0)),
            pl.BlockSpec((d_pn, h_dim), lambda i: (0, 0)),
            pl.BlockSpec((d_es, h_dim), lambda i: (0, 0)),
            pl.BlockSpec((1, h_dim), lambda i: (0, 0)),
            pl.BlockSpec((1, 1), lambda i: (0, 0)),
            pl.BlockSpec((h_dim, d_out), lambda i: (0, 0)),
            pl.BlockSpec((1, d_out), lambda i: (0, 0)),
        ],
        out_specs=[
            pl.BlockSpec((blk, 2 * d_out), lambda i: (i, 0)),
            pl.BlockSpec((blk, d_out), lambda i: (i, 0)),
            pl.BlockSpec((blk, d_out), lambda i: (i, 0)),
        ],
        out_shape=[
            jax.ShapeDtypeStruct((n, 2 * d_out), jnp.float32),
            jax.ShapeDtypeStruct((n, d_out), jnp.float32),
            jax.ShapeDtypeStruct((n, d_out), jnp.float32),
        ],
    )(pn_c, ns_c, es_c,
      a_pn, a_ns, a_es, bn0.reshape(1, h_dim), an0.reshape(1, 1),
      Wnf.T, bnf.reshape(1, d_out))


def _edge_mlp_body(pe_ref, vi_ref, vj_ref,
                   b_pe, b_vi, b_vj, b0, alpha, wf, bf, out_r, out_i):
    bft = jnp.bfloat16
    pe_full = pe_ref[...].astype(bft)
    vi_full = vi_ref[...].astype(bft)
    vj_full = vj_ref[...].astype(bft)
    d_pe = pe_full.shape[1] // 2
    d_v = vi_full.shape[1] // 2
    for k, out in enumerate((out_r, out_i)):
        pe = pe_full[:, k * d_pe:(k + 1) * d_pe]
        vi = vi_full[:, k * d_v:(k + 1) * d_v]
        vj = vj_full[:, k * d_v:(k + 1) * d_v]
        g = (jnp.dot(pe, b_pe[...].astype(bft), preferred_element_type=jnp.float32)
             + jnp.dot(vi, b_vi[...].astype(bft), preferred_element_type=jnp.float32)
             + jnp.dot(vj, b_vj[...].astype(bft), preferred_element_type=jnp.float32)
             + b0[...])
        g = jnp.where(g >= 0, g, alpha[...] * g)
        out[...] = jnp.dot(g, wf[...], preferred_element_type=jnp.float32) + bf[...]


def _edge_mlp(pe_c, vi_c, vj_c, We0, be0, ae0, Wef, bef, blk):
    e = pe_c.shape[0]
    d_pe = pe_c.shape[1] // 2
    d_v = vi_c.shape[1] // 2
    h_dim = We0.shape[0]
    d_out = Wef.shape[0]
    grid = e // blk
    W0t = We0.T
    b_pe = W0t[:d_pe]
    b_vi = W0t[d_pe:d_pe + d_v]
    b_vj = W0t[d_pe + d_v:]
    out_sds = jax.ShapeDtypeStruct((e, d_out), jnp.float32)
    return pl.pallas_call(
        _edge_mlp_body,
        grid=(grid,),
        in_specs=[
            pl.BlockSpec((blk, 2 * d_pe), lambda i: (i, 0)),
            pl.BlockSpec((blk, 2 * d_v), lambda i: (i, 0)),
            pl.BlockSpec((blk, 2 * d_v), lambda i: (i, 0)),
            pl.BlockSpec((d_pe, h_dim), lambda i: (0, 0)),
            pl.BlockSpec((d_v, h_dim), lambda i: (0, 0)),
            pl.BlockSpec((d_v, h_dim), lambda i: (0, 0)),
            pl.BlockSpec((1, h_dim), lambda i: (0, 0)),
            pl.BlockSpec((1, 1), lambda i: (0, 0)),
            pl.BlockSpec((h_dim, d_out), lambda i: (0, 0)),
            pl.BlockSpec((1, d_out), lambda i: (0, 0)),
        ],
        out_specs=[
            pl.BlockSpec((blk, d_out), lambda i: (i, 0)),
            pl.BlockSpec((blk, d_out), lambda i: (i, 0)),
        ],
        out_shape=[out_sds, out_sds],
    )(pe_c, vi_c, vj_c,
      b_pe, b_vi, b_vj, be0.reshape(1, h_dim), ae0.reshape(1, 1),
      Wef.T, bef.reshape(1, d_out))


def _sc_mesh():
    return plsc.VectorSubcoreMesh(core_axis_name="c", subcore_axis_name="s",
                                  num_cores=NC, num_subcores=NS)


def _node_seg_sum_sc(row3, col2, pn_c, n_pad, rows_per_sub, e):
    ew = e // NW
    nchunk = ew // CHUNK
    d_n = pn_c.shape[1]
    zrows = rows_per_sub // 8

    def body(row_hbm, col_hbm, pn_hbm, ns_hbm,
             row_v, col_v, grow0, grow1, zbuf_n, acc_n,
             gsem0, gsem1):
        cid = lax.axis_index("c")
        sid = lax.axis_index("s")
        wid = sid * NC + cid
        grows = (grow0, grow1)
        gsems = (gsem0, gsem1)

        zero16 = jnp.zeros((16,), jnp.float32)
        for r in range(8):
            for cc in range(d_n // 16):
                zbuf_n[r, pl.ds(cc * 16, 16)] = zero16
        r0 = sid * rows_per_sub

        def zcopy(z, _):
            pltpu.sync_copy(zbuf_n, acc_n.at[pl.ds(r0 + z * 8, 8)])
            return 0

        lax.fori_loop(0, zrows, zcopy, 0)
        plsc.subcore_barrier()

        pltpu.sync_copy(row_hbm.at[wid], row_v)
        pltpu.sync_copy(col_hbm.at[wid], col_v)

        def gather_fire(j, p):
            idx = col_v.at[pl.ds(j * CHUNK, CHUNK)]
            pltpu.async_copy(pn_hbm.at[idx], grows[p], gsems[p])

        for p in range(2):
            gather_fire(p, p)

        def group(g, _):
            for p in range(2):
                j = g * 2 + p
                pltpu.make_async_copy(pn_hbm.at[pl.ds(0, CHUNK)],
                                      grows[p], gsems[p]).wait()
                pltpu.sync_copy(grows[p], acc_n.at[row_v.at[j]], add=True)

                @pl.when(j + 2 < nchunk)
                def _gf():
                    gather_fire(j + 2, p)

            return 0

        lax.fori_loop(0, nchunk // 2, group, 0)
        for j in range(nchunk - nchunk % 2, nchunk):
            p = j % 2
            pltpu.make_async_copy(pn_hbm.at[pl.ds(0, CHUNK)],
                                  grows[p], gsems[p]).wait()
            pltpu.sync_copy(grows[p], acc_n.at[row_v.at[j]], add=True)
        plsc.subcore_barrier()

        pltpu.sync_copy(acc_n.at[pl.ds(r0, rows_per_sub)],
                        ns_hbm.at[cid, pl.ds(r0, rows_per_sub)])

    f = pl.kernel(
        body,
        out_type=jax.ShapeDtypeStruct((NC, n_pad, d_n), jnp.float32),
        mesh=_sc_mesh(),
        scratch_types=[
            pltpu.VMEM((nchunk, CHUNK), jnp.int32),
            pltpu.VMEM((ew,), jnp.int32),
            pltpu.VMEM((CHUNK, d_n), jnp.float32),
            pltpu.VMEM((CHUNK, d_n), jnp.float32),
            pltpu.VMEM((8, d_n), jnp.float32),
            pltpu.VMEM_SHARED((n_pad, d_n), jnp.float32),
            pltpu.SemaphoreType.DMA,
            pltpu.SemaphoreType.DMA,
        ],
    )
    return f(row3, col2, pn_c)


def _edge_seg_sum_sc(row3, pe_c, n_pad, rows_per_sub):
    e = pe_c.shape[0]
    ew = e // NW
    nchunk = ew // CHUNK
    d_e = pe_c.shape[1]
    d_w = 128

    def body(row_hbm, pe_hbm, es_hbm,
             row_v, pe_s, pe_v, zbuf_e, acc_e, esem):
        cid = lax.axis_index("c")
        sid = lax.axis_index("s")
        wid = sid * NC + cid

        zero16 = jnp.zeros((16,), jnp.float32)
        for r in range(8):
            for cc in range(d_w // 16):
                zbuf_e[r, pl.ds(cc * 16, 16)] = zero16
        for r in range(CHUNK):
            for cc in range(d_w // 16):
                pe_v[r, pl.ds(cc * 16, 16)] = zero16
        r0 = sid * rows_per_sub

        def zcopy(z, _):
            pltpu.sync_copy(zbuf_e, acc_e.at[pl.ds(r0 + z * 8, 8)])
            return 0

        lax.fori_loop(0, rows_per_sub // 8, zcopy, 0)
        plsc.subcore_barrier()

        ebase = wid * ew
        pltpu.sync_copy(row_hbm.at[wid], row_v)

        def pe_fire(j):
            pltpu.async_copy(pe_hbm.at[pl.ds(ebase + j * CHUNK, CHUNK)],
                             pe_s, esem)

        pe_fire(0)

        def step(j, _):
            pltpu.make_async_copy(pe_hbm.at[pl.ds(0, CHUNK)],
                                  pe_s, esem).wait()
            for r in range(CHUNK):
                for t in range(d_e // 16):
                    pe_v[r, pl.ds(t * 16, 16)] = pe_s[r, pl.ds(t * 16, 16)]

            @pl.when(j + 1 < nchunk)
            def _nf():
                pe_fire(j + 1)

            pltpu.sync_copy(pe_v, acc_e.at[row_v.at[j]], add=True)
            return 0

        lax.fori_loop(0, nchunk, step, 0)
        plsc.subcore_barrier()

        pltpu.sync_copy(acc_e.at[pl.ds(r0, rows_per_sub)],
                        es_hbm.at[cid, pl.ds(r0, rows_per_sub)])

    f = pl.kernel(
        body,
        out_type=jax.ShapeDtypeStruct((NC, n_pad, d_w), jnp.float32),
        mesh=_sc_mesh(),
        scratch_types=[
            pltpu.VMEM((nchunk, CHUNK), jnp.int32),
            pltpu.VMEM((CHUNK, d_e), jnp.float32),
            pltpu.VMEM((CHUNK, d_w), jnp.float32),
            pltpu.VMEM((8, d_w), jnp.float32),
            pltpu.VMEM_SHARED((n_pad, d_w), jnp.float32),
            pltpu.SemaphoreType.DMA,
        ],
    )
    return f(row3, pe_c)


def _edge_gather_sc(row2, col2, no_c, e):
    ew = e // NW
    nchunk = ew // CHUNK
    d_n = no_c.shape[1]
    dt = no_c.dtype

    def body(row_hbm, col_hbm, no_hbm, vi_hbm, vj_hbm,
             row_v, col_v, a0, a1, b0, b1,
             sa0, sa1, sb0, sb1):
        cid = lax.axis_index("c")
        sid = lax.axis_index("s")
        wid = sid * NC + cid
        abufs, bbufs = (a0, a1), (b0, b1)
        asems, bsems = (sa0, sa1), (sb0, sb1)
        ebase = wid * ew

        pltpu.sync_copy(row_hbm.at[wid], row_v)
        pltpu.sync_copy(col_hbm.at[wid], col_v)

        def gather_fire(j, p):
            sl = pl.ds(j * CHUNK, CHUNK)
            pltpu.async_copy(no_hbm.at[row_v.at[sl]], abufs[p], asems[p])
            pltpu.async_copy(no_hbm.at[col_v.at[sl]], bbufs[p], bsems[p])

        for p in range(2):
            gather_fire(p, p)

        def drain_write(j, p):
            dst = pl.ds(ebase + j * CHUNK, CHUNK)
            pltpu.make_async_copy(no_hbm.at[pl.ds(0, CHUNK)],
                                  abufs[p], asems[p]).wait()
            pltpu.sync_copy(abufs[p], vi_hbm.at[dst])
            pltpu.make_async_copy(no_hbm.at[pl.ds(0, CHUNK)],
                                  bbufs[p], bsems[p]).wait()
            pltpu.sync_copy(bbufs[p], vj_hbm.at[dst])

        def group(g, _):
            for p in range(2):
                j = g * 2 + p
                drain_write(j, p)

                @pl.when(j + 2 < nchunk)
                def _gf():
                    gather_fire(j + 2, p)

            return 0

        lax.fori_loop(0, nchunk // 2, group, 0)
        for j in range(nchunk - nchunk % 2, nchunk):
            drain_write(j, j % 2)

    out_sds = jax.ShapeDtypeStruct((e, d_n), dt)
    f = pl.kernel(
        body,
        out_type=(out_sds, out_sds),
        mesh=_sc_mesh(),
        scratch_types=[
            pltpu.VMEM((ew,), jnp.int32),
            pltpu.VMEM((ew,), jnp.int32),
            pltpu.VMEM((CHUNK, d_n), dt),
            pltpu.VMEM((CHUNK, d_n), dt),
            pltpu.VMEM((CHUNK, d_n), dt),
            pltpu.VMEM((CHUNK, d_n), dt),
            pltpu.SemaphoreType.DMA,
            pltpu.SemaphoreType.DMA,
            pltpu.SemaphoreType.DMA,
            pltpu.SemaphoreType.DMA,
        ],
    )
    return f(row2, col2, no_c)


def kernel(node_feats_real, node_feats_imag, edge_feats_real, edge_feats_imag,
           edge_index, Wpn, bpn, Wpe, bpe, Wn0, bn0, an0, Wnf, bnf,
           We0, be0, ae0, Wef, bef):
    n = node_feats_real.shape[0]
    e = edge_feats_real.shape[0]
    d_out_node = Wnf.shape[0]

    ew = e // NW
    assert ew * NW == e and ew % CHUNK == 0
    nchunk = ew // CHUNK
    rows_per_sub = (-(-n // NS) + 7) // 8 * 8
    n_pad = rows_per_sub * NS

    row = edge_index[:, 0]
    col = edge_index[:, 1]
    row3 = row.reshape(NW, nchunk, CHUNK)
    row2 = row.reshape(NW, ew)
    col2 = col.reshape(NW, ew)

    pn_c = _projection(node_feats_real, node_feats_imag, Wpn, bpn, blk=1000)
    pe_c = _projection(edge_feats_real, edge_feats_imag, Wpe, bpe, blk=4000)

    ns_c = _node_seg_sum_sc(row3, col2, pn_c, n_pad, rows_per_sub, e)
    es_c = _edge_seg_sum_sc(row3, pe_c, n_pad, rows_per_sub)

    no_c, no_r, no_i = _node_mlp(pn_c, ns_c, es_c, Wn0, bn0, an0, Wnf, bnf,
                                 blk=1000)

    vi_c, vj_c = _edge_gather_sc(row2, col2, no_c, e)

    eo_r, eo_i = _edge_mlp(pe_c, vi_c, vj_c, We0, be0, ae0, Wef, bef, blk=8000)

    return no_r, no_i, eo_r, eo_i
